# Initial kernel scaffold; baseline (speedup 1.0000x reference)
import jax
import jax.numpy as jnp
from jax.experimental import pallas as pl


def kernel(x, edge_index, W_nei0, W_self0, b0, g0, be0, W_nei1, W_self1, b1, g1, be1, fcW, fcb):
    raise NotImplementedError("write your pallas kernel here")


# speedup vs baseline: 6.2648x; 6.2648x over previous
import functools

import jax
import jax.numpy as jnp
from jax import lax
from jax.experimental import pallas as pl
from jax.experimental.pallas import tpu as pltpu
from jax.experimental.pallas import tpu_sc as plsc

N_NODES = 10000
N_EDGES = 320000
D_FEAT = 128
D_HID = 256
N_CLASS = 47

NC = 2
NS = 16
NW = NC * NS
EC = 80
ET = N_EDGES // NW
CT = ET // EC
STRIPE_STEP = 624
STRIPE_LEN = 640


_MESH = plsc.VectorSubcoreMesh(
    core_axis_name="c", subcore_axis_name="s", num_cores=NC, num_subcores=NS)


@functools.partial(
    pl.kernel, mesh=_MESH,
    out_type=jax.ShapeDtypeStruct((NC * N_NODES, D_FEAT), jnp.float32),
    scratch_types=[
        pltpu.VMEM((ET,), jnp.int32),
        pltpu.VMEM((CT, EC), jnp.int32),
        pltpu.VMEM((EC, D_FEAT), jnp.float32),
        pltpu.VMEM_SHARED((N_NODES, D_FEAT), jnp.float32),
        pltpu.SemaphoreType.DMA,
    ])
def _sc_agg(table, src_hbm, dst_hbm, z_feat, out_hbm,
            src_v, dst_v, rows_v, acc, sem):
  c = lax.axis_index("c")
  s = lax.axis_index("s")
  wid = c * NS + s

  pltpu.sync_copy(src_hbm.at[pl.ds(wid * ET, ET)], src_v)
  pltpu.sync_copy(dst_hbm.at[wid], dst_v)

  stripe = pl.ds(s * STRIPE_STEP, STRIPE_LEN)
  pltpu.sync_copy(z_feat.at[stripe], acc.at[stripe])
  plsc.subcore_barrier()

  def chunk(i, carry):
    idx = src_v.at[pl.ds(i * EC, EC)]
    pltpu.async_copy(table.at[idx], rows_v, sem).wait()
    pltpu.sync_copy(rows_v, acc.at[dst_v.at[i]], add=True)
    return carry

  lax.fori_loop(0, CT, chunk, 0)
  plsc.subcore_barrier()

  out_rows = pl.ds(c * N_NODES + s * STRIPE_STEP, STRIPE_LEN)
  pltpu.sync_copy(acc.at[stripe], out_hbm.at[out_rows])


@functools.partial(
    pl.kernel, mesh=_MESH,
    out_type=jax.ShapeDtypeStruct((NW, N_NODES), jnp.float32),
    compiler_params=pltpu.CompilerParams(needs_layout_passes=False),
    scratch_types=[
        pltpu.VMEM((ET,), jnp.int32),
        pltpu.VMEM((N_NODES,), jnp.float32),
    ])
def _sc_counts(dst_hbm, z1d, cnt_out_hbm, dst_v, hist_v):
  c = lax.axis_index("c")
  s = lax.axis_index("s")
  wid = c * NS + s

  pltpu.sync_copy(dst_hbm.at[pl.ds(wid * ET, ET)], dst_v)
  pltpu.sync_copy(z1d, hist_v)

  ones = jnp.ones((16,), jnp.float32)

  def step(i, carry):
    idx = dst_v[pl.ds(i * 16, 16)]
    plsc.addupdate_scatter(hist_v, [idx], ones)
    return carry

  lax.fori_loop(0, ET // 16, step, 0)
  pltpu.sync_copy(hist_v, cnt_out_hbm.at[wid])


def _tc_cnt_reduce(cntp):
  def body(cntp_r, out_r):
    out_r[:] = jnp.sum(cntp_r[:], axis=0, keepdims=True)
  return pl.pallas_call(
      body,
      out_shape=jax.ShapeDtypeStruct((1, N_NODES), jnp.float32),
  )(cntp)


_BLK = 1000
_GRID = N_NODES // _BLK


def _dense1_body(x, aggp, cnt, wn, ws, b, g, be, h1a, h1b):
  agg = aggp[0] + aggp[1]
  mean = agg / jnp.maximum(cnt, 1.0)
  pre = (jnp.dot(mean, wn[:], preferred_element_type=jnp.float32)
         + jnp.dot(x[:], ws[:], preferred_element_type=jnp.float32) + b[:])
  mu = jnp.mean(pre, axis=-1, keepdims=True)
  var = jnp.mean((pre - mu) ** 2, axis=-1, keepdims=True)
  h = (pre - mu) * lax.rsqrt(var + 1e-5) * g[:] + be[:]
  h = jnp.maximum(h, 0.0)
  h1a[:] = h[:, :D_FEAT]
  h1b[:] = h[:, D_FEAT:]


def _tc_dense1(x, aggp, cnt, wn, ws, b, g, be):
  def body(x_r, aggp_r, cnt_r, wn_r, ws_r, b_r, g_r, be_r, h1a_r, h1b_r):
    _dense1_body(x_r[:], aggp_r[:], cnt_r[:], wn_r, ws_r, b_r, g_r, be_r,
                 h1a_r, h1b_r)
  return pl.pallas_call(
      body,
      grid=(_GRID,),
      in_specs=[
          pl.BlockSpec((_BLK, D_FEAT), lambda i: (i, 0)),
          pl.BlockSpec((NC, _BLK, D_FEAT), lambda i: (0, i, 0)),
          pl.BlockSpec((_BLK, 1), lambda i: (i, 0)),
          pl.BlockSpec((D_FEAT, D_HID), lambda i: (0, 0)),
          pl.BlockSpec((D_FEAT, D_HID), lambda i: (0, 0)),
          pl.BlockSpec((1, D_HID), lambda i: (0, 0)),
          pl.BlockSpec((1, D_HID), lambda i: (0, 0)),
          pl.BlockSpec((1, D_HID), lambda i: (0, 0)),
      ],
      out_specs=[
          pl.BlockSpec((_BLK, D_FEAT), lambda i: (i, 0)),
          pl.BlockSpec((_BLK, D_FEAT), lambda i: (i, 0)),
      ],
      out_shape=[
          jax.ShapeDtypeStruct((N_NODES, D_FEAT), jnp.float32),
          jax.ShapeDtypeStruct((N_NODES, D_FEAT), jnp.float32),
      ],
  )(x, aggp, cnt, wn, ws, b, g, be)


def _tc_dense2(h1a, h1b, aggap, aggbp, cnt, wn, ws, b, g, be, fcw, fcb):
  def body(h1a_r, h1b_r, aggap_r, aggbp_r, cnt_r, wn_r, ws_r, b_r, g_r,
           be_r, fcw_r, fcb_r, out_r):
    h1 = jnp.concatenate([h1a_r[:], h1b_r[:]], axis=1)
    agg = jnp.concatenate(
        [aggap_r[0] + aggap_r[1], aggbp_r[0] + aggbp_r[1]], axis=1)
    mean = agg / jnp.maximum(cnt_r[:], 1.0)
    pre = (jnp.dot(mean, wn_r[:], preferred_element_type=jnp.float32)
           + jnp.dot(h1, ws_r[:], preferred_element_type=jnp.float32)
           + b_r[:])
    mu = jnp.mean(pre, axis=-1, keepdims=True)
    var = jnp.mean((pre - mu) ** 2, axis=-1, keepdims=True)
    h = (pre - mu) * lax.rsqrt(var + 1e-5) * g_r[:] + be_r[:]
    h = jnp.maximum(h, 0.0)
    res = jnp.dot(h, fcw_r[:], preferred_element_type=jnp.float32) + fcb_r[:]
    out_r[:] = res[:, :N_CLASS]

  return pl.pallas_call(
      body,
      grid=(_GRID,),
      in_specs=[
          pl.BlockSpec((_BLK, D_FEAT), lambda i: (i, 0)),
          pl.BlockSpec((_BLK, D_FEAT), lambda i: (i, 0)),
          pl.BlockSpec((NC, _BLK, D_FEAT), lambda i: (0, i, 0)),
          pl.BlockSpec((NC, _BLK, D_FEAT), lambda i: (0, i, 0)),
          pl.BlockSpec((_BLK, 1), lambda i: (i, 0)),
          pl.BlockSpec((D_HID, D_HID), lambda i: (0, 0)),
          pl.BlockSpec((D_HID, D_HID), lambda i: (0, 0)),
          pl.BlockSpec((1, D_HID), lambda i: (0, 0)),
          pl.BlockSpec((1, D_HID), lambda i: (0, 0)),
          pl.BlockSpec((1, D_HID), lambda i: (0, 0)),
          pl.BlockSpec((D_HID, 128), lambda i: (0, 0)),
          pl.BlockSpec((1, 128), lambda i: (0, 0)),
      ],
      out_specs=pl.BlockSpec((_BLK, N_CLASS), lambda i: (i, 0)),
      out_shape=jax.ShapeDtypeStruct((N_NODES, N_CLASS), jnp.float32),
  )(h1a, h1b, aggap, aggbp, cnt, wn, ws, b, g, be, fcw, fcb)


def kernel(x, edge_index, W_nei0, W_self0, b0, g0, be0,
           W_nei1, W_self1, b1, g1, be1, fcW, fcb):
  dst = edge_index[0]
  src = edge_index[1]
  dst2 = dst.reshape(NW, CT, EC)
  z_feat = jnp.zeros((N_NODES, D_FEAT), jnp.float32)
  z1d = jnp.zeros((N_NODES,), jnp.float32)

  agg0p = _sc_agg(x, src, dst2, z_feat).reshape(NC, N_NODES, D_FEAT)
  cntp = _sc_counts(dst, z1d)
  cnt = _tc_cnt_reduce(cntp).reshape(N_NODES, 1)

  h1a, h1b = _tc_dense1(x, agg0p, cnt, W_nei0, W_self0,
                        b0.reshape(1, D_HID), g0.reshape(1, D_HID),
                        be0.reshape(1, D_HID))

  agg1ap = _sc_agg(h1a, src, dst2, z_feat).reshape(NC, N_NODES, D_FEAT)
  agg1bp = _sc_agg(h1b, src, dst2, z_feat).reshape(NC, N_NODES, D_FEAT)

  fcw_pad = jnp.zeros((D_HID, 128), jnp.float32).at[:, :N_CLASS].set(fcW)
  fcb_pad = jnp.zeros((1, 128), jnp.float32).at[:, :N_CLASS].set(fcb)

  return _tc_dense2(h1a, h1b, agg1ap, agg1bp, cnt, W_nei1, W_self1,
                    b1.reshape(1, D_HID), g1.reshape(1, D_HID),
                    be1.reshape(1, D_HID), fcw_pad, fcb_pad)


# speedup vs baseline: 10.0068x; 1.5973x over previous
import functools

import jax
import jax.numpy as jnp
from jax import lax
from jax.experimental import pallas as pl
from jax.experimental.pallas import tpu as pltpu
from jax.experimental.pallas import tpu_sc as plsc

N_NODES = 10000
N_EDGES = 320000
D_FEAT = 128
D_HID = 256
N_CLASS = 47

NC = 2
NS = 16
NW = NC * NS
EC = 80
ET = N_EDGES // NW
CT = ET // EC
STRIPE_STEP = 624
STRIPE_LEN = 640


_MESH = plsc.VectorSubcoreMesh(
    core_axis_name="c", subcore_axis_name="s", num_cores=NC, num_subcores=NS)


@functools.partial(
    pl.kernel, mesh=_MESH,
    out_type=jax.ShapeDtypeStruct((NC * N_NODES, D_FEAT), jnp.float32),
    scratch_types=[
        pltpu.VMEM((ET,), jnp.int32),
        pltpu.VMEM((CT, EC), jnp.int32),
        pltpu.VMEM((EC, D_FEAT), jnp.float32),
        pltpu.VMEM((EC, D_FEAT), jnp.float32),
        pltpu.VMEM_SHARED((N_NODES, D_FEAT), jnp.float32),
        pltpu.SemaphoreType.DMA,
        pltpu.SemaphoreType.DMA,
    ])
def _sc_agg(table, src_hbm, dst_hbm, z_feat, out_hbm,
            src_v, dst_v, rows0, rows1, acc, gsem, ssem):
  c = lax.axis_index("c")
  s = lax.axis_index("s")
  wid = c * NS + s

  pltpu.sync_copy(src_hbm.at[pl.ds(wid * ET, ET)], src_v)
  pltpu.sync_copy(dst_hbm.at[wid], dst_v)

  stripe = pl.ds(s * STRIPE_STEP, STRIPE_LEN)
  pltpu.sync_copy(z_feat.at[stripe], acc.at[stripe])
  plsc.subcore_barrier()

  def gstart(i, buf):
    pltpu.async_copy(table.at[src_v.at[pl.ds(i * EC, EC)]], buf, gsem)

  def drain(sem, buf):
    pltpu.make_async_copy(z_feat.at[pl.ds(0, EC)], buf, sem).wait()

  def step(i, cur, oth):
    @pl.when(i > 0)
    def _():
      drain(ssem, oth)
    @pl.when(i + 1 < CT)
    def _():
      gstart(i + 1, oth)
    drain(gsem, cur)
    pltpu.async_copy(cur, acc.at[dst_v.at[i]], ssem, add=True)

  gstart(0, rows0)

  def pair(j, carry):
    step(2 * j, rows0, rows1)
    step(2 * j + 1, rows1, rows0)
    return carry

  lax.fori_loop(0, CT // 2, pair, 0)
  if CT % 2:
    step(CT - 1, rows0, rows1)
    drain(ssem, rows0)
  else:
    drain(ssem, rows1)
  plsc.subcore_barrier()

  out_rows = pl.ds(c * N_NODES + s * STRIPE_STEP, STRIPE_LEN)
  pltpu.sync_copy(acc.at[stripe], out_hbm.at[out_rows])


@functools.partial(
    pl.kernel, mesh=_MESH,
    out_type=jax.ShapeDtypeStruct((NW, N_NODES), jnp.float32),
    compiler_params=pltpu.CompilerParams(needs_layout_passes=False),
    scratch_types=[
        pltpu.VMEM((ET,), jnp.int32),
        pltpu.VMEM((N_NODES,), jnp.float32),
    ])
def _sc_counts(dst_hbm, z1d, cnt_out_hbm, dst_v, hist_v):
  c = lax.axis_index("c")
  s = lax.axis_index("s")
  wid = c * NS + s

  pltpu.sync_copy(dst_hbm.at[pl.ds(wid * ET, ET)], dst_v)
  pltpu.sync_copy(z1d, hist_v)

  ones = jnp.ones((16,), jnp.float32)

  def step(i, carry):
    idx = dst_v[pl.ds(i * 16, 16)]
    plsc.addupdate_scatter(hist_v, [idx], ones)
    return carry

  lax.fori_loop(0, ET // 16, step, 0)
  pltpu.sync_copy(hist_v, cnt_out_hbm.at[wid])


def _tc_cnt_reduce(cntp):
  def body(cntp_r, out_r):
    out_r[:] = jnp.sum(cntp_r[:], axis=0, keepdims=True)
  return pl.pallas_call(
      body,
      out_shape=jax.ShapeDtypeStruct((1, N_NODES), jnp.float32),
  )(cntp)


_BLK = 1000
_GRID = N_NODES // _BLK


def _dense1_body(x, aggp, cnt, wn, ws, b, g, be, h1a, h1b):
  agg = aggp[0] + aggp[1]
  mean = agg / jnp.maximum(cnt, 1.0)
  pre = (jnp.dot(mean, wn[:], preferred_element_type=jnp.float32)
         + jnp.dot(x[:], ws[:], preferred_element_type=jnp.float32) + b[:])
  mu = jnp.mean(pre, axis=-1, keepdims=True)
  var = jnp.mean((pre - mu) ** 2, axis=-1, keepdims=True)
  h = (pre - mu) * lax.rsqrt(var + 1e-5) * g[:] + be[:]
  h = jnp.maximum(h, 0.0)
  h1a[:] = h[:, :D_FEAT]
  h1b[:] = h[:, D_FEAT:]


def _tc_dense1(x, aggp, cnt, wn, ws, b, g, be):
  def body(x_r, aggp_r, cnt_r, wn_r, ws_r, b_r, g_r, be_r, h1a_r, h1b_r):
    _dense1_body(x_r[:], aggp_r[:], cnt_r[:], wn_r, ws_r, b_r, g_r, be_r,
                 h1a_r, h1b_r)
  return pl.pallas_call(
      body,
      grid=(_GRID,),
      in_specs=[
          pl.BlockSpec((_BLK, D_FEAT), lambda i: (i, 0)),
          pl.BlockSpec((NC, _BLK, D_FEAT), lambda i: (0, i, 0)),
          pl.BlockSpec((_BLK, 1), lambda i: (i, 0)),
          pl.BlockSpec((D_FEAT, D_HID), lambda i: (0, 0)),
          pl.BlockSpec((D_FEAT, D_HID), lambda i: (0, 0)),
          pl.BlockSpec((1, D_HID), lambda i: (0, 0)),
          pl.BlockSpec((1, D_HID), lambda i: (0, 0)),
          pl.BlockSpec((1, D_HID), lambda i: (0, 0)),
      ],
      out_specs=[
          pl.BlockSpec((_BLK, D_FEAT), lambda i: (i, 0)),
          pl.BlockSpec((_BLK, D_FEAT), lambda i: (i, 0)),
      ],
      out_shape=[
          jax.ShapeDtypeStruct((N_NODES, D_FEAT), jnp.float32),
          jax.ShapeDtypeStruct((N_NODES, D_FEAT), jnp.float32),
      ],
  )(x, aggp, cnt, wn, ws, b, g, be)


def _tc_dense2(h1a, h1b, aggap, aggbp, cnt, wn, ws, b, g, be, fcw, fcb):
  def body(h1a_r, h1b_r, aggap_r, aggbp_r, cnt_r, wn_r, ws_r, b_r, g_r,
           be_r, fcw_r, fcb_r, out_r):
    h1 = jnp.concatenate([h1a_r[:], h1b_r[:]], axis=1)
    agg = jnp.concatenate(
        [aggap_r[0] + aggap_r[1], aggbp_r[0] + aggbp_r[1]], axis=1)
    mean = agg / jnp.maximum(cnt_r[:], 1.0)
    pre = (jnp.dot(mean, wn_r[:], preferred_element_type=jnp.float32)
           + jnp.dot(h1, ws_r[:], preferred_element_type=jnp.float32)
           + b_r[:])
    mu = jnp.mean(pre, axis=-1, keepdims=True)
    var = jnp.mean((pre - mu) ** 2, axis=-1, keepdims=True)
    h = (pre - mu) * lax.rsqrt(var + 1e-5) * g_r[:] + be_r[:]
    h = jnp.maximum(h, 0.0)
    res = jnp.dot(h, fcw_r[:], preferred_element_type=jnp.float32) + fcb_r[:]
    out_r[:] = res[:, :N_CLASS]

  return pl.pallas_call(
      body,
      grid=(_GRID,),
      in_specs=[
          pl.BlockSpec((_BLK, D_FEAT), lambda i: (i, 0)),
          pl.BlockSpec((_BLK, D_FEAT), lambda i: (i, 0)),
          pl.BlockSpec((NC, _BLK, D_FEAT), lambda i: (0, i, 0)),
          pl.BlockSpec((NC, _BLK, D_FEAT), lambda i: (0, i, 0)),
          pl.BlockSpec((_BLK, 1), lambda i: (i, 0)),
          pl.BlockSpec((D_HID, D_HID), lambda i: (0, 0)),
          pl.BlockSpec((D_HID, D_HID), lambda i: (0, 0)),
          pl.BlockSpec((1, D_HID), lambda i: (0, 0)),
          pl.BlockSpec((1, D_HID), lambda i: (0, 0)),
          pl.BlockSpec((1, D_HID), lambda i: (0, 0)),
          pl.BlockSpec((D_HID, 128), lambda i: (0, 0)),
          pl.BlockSpec((1, 128), lambda i: (0, 0)),
      ],
      out_specs=pl.BlockSpec((_BLK, N_CLASS), lambda i: (i, 0)),
      out_shape=jax.ShapeDtypeStruct((N_NODES, N_CLASS), jnp.float32),
  )(h1a, h1b, aggap, aggbp, cnt, wn, ws, b, g, be, fcw, fcb)


def kernel(x, edge_index, W_nei0, W_self0, b0, g0, be0,
           W_nei1, W_self1, b1, g1, be1, fcW, fcb):
  dst = edge_index[0]
  src = edge_index[1]
  dst2 = dst.reshape(NW, CT, EC)
  z_feat = jnp.zeros((N_NODES, D_FEAT), jnp.float32)
  z1d = jnp.zeros((N_NODES,), jnp.float32)

  agg0p = _sc_agg(x, src, dst2, z_feat).reshape(NC, N_NODES, D_FEAT)
  cntp = _sc_counts(dst, z1d)
  cnt = _tc_cnt_reduce(cntp).reshape(N_NODES, 1)

  h1a, h1b = _tc_dense1(x, agg0p, cnt, W_nei0, W_self0,
                        b0.reshape(1, D_HID), g0.reshape(1, D_HID),
                        be0.reshape(1, D_HID))

  agg1ap = _sc_agg(h1a, src, dst2, z_feat).reshape(NC, N_NODES, D_FEAT)
  agg1bp = _sc_agg(h1b, src, dst2, z_feat).reshape(NC, N_NODES, D_FEAT)

  fcw_pad = jnp.zeros((D_HID, 128), jnp.float32).at[:, :N_CLASS].set(fcW)
  fcb_pad = jnp.zeros((1, 128), jnp.float32).at[:, :N_CLASS].set(fcb)

  return _tc_dense2(h1a, h1b, agg1ap, agg1bp, cnt, W_nei1, W_self1,
                    b1.reshape(1, D_HID), g1.reshape(1, D_HID),
                    be1.reshape(1, D_HID), fcw_pad, fcb_pad)


# speedup vs baseline: 10.4280x; 1.0421x over previous
import functools

import jax
import jax.numpy as jnp
from jax import lax
from jax.experimental import pallas as pl
from jax.experimental.pallas import tpu as pltpu
from jax.experimental.pallas import tpu_sc as plsc

N_NODES = 10000
N_EDGES = 320000
D_FEAT = 128
D_HID = 256
N_CLASS = 47

NC = 2
NS = 16
NW = NC * NS
EC = 80
ET = N_EDGES // NW
CT = ET // EC
STRIPE_STEP = 624
STRIPE_LEN = 640


_MESH = plsc.VectorSubcoreMesh(
    core_axis_name="c", subcore_axis_name="s", num_cores=NC, num_subcores=NS)


def _pipeline(table, src_v, dst_v, acc, rows0, rows1, gsem, ssem, z_feat,
              ct, hist=None):
  def gstart(i, buf):
    pltpu.async_copy(table.at[src_v.at[pl.ds(i * EC, EC)]], buf, gsem)

  def drain(sem, buf):
    pltpu.make_async_copy(z_feat.at[pl.ds(0, EC)], buf, sem).wait()

  def step(i, cur, oth):
    @pl.when(i > 0)
    def _():
      drain(ssem, oth)
    @pl.when(i + 1 < ct)
    def _():
      gstart(i + 1, oth)
    if hist is not None:
      ones = jnp.ones((16,), jnp.float32)
      for k in range(EC // 16):
        idx = dst_v[i, pl.ds(k * 16, 16)]
        plsc.addupdate_scatter(hist, [idx], ones)
    drain(gsem, cur)
    pltpu.async_copy(cur, acc.at[dst_v.at[i]], ssem, add=True)

  gstart(0, rows0)

  def pair(j, carry):
    step(2 * j, rows0, rows1)
    step(2 * j + 1, rows1, rows0)
    return carry

  lax.fori_loop(0, ct // 2, pair, 0)
  if ct % 2:
    step(ct - 1, rows0, rows1)
    drain(ssem, rows0)
  else:
    drain(ssem, rows1)


@functools.partial(
    pl.kernel, mesh=_MESH,
    out_type=jax.ShapeDtypeStruct((NC * N_NODES, D_FEAT), jnp.float32),
    scratch_types=[
        pltpu.VMEM((ET,), jnp.int32),
        pltpu.VMEM((CT, EC), jnp.int32),
        pltpu.VMEM((EC, D_FEAT), jnp.float32),
        pltpu.VMEM((EC, D_FEAT), jnp.float32),
        pltpu.VMEM_SHARED((N_NODES, D_FEAT), jnp.float32),
        pltpu.SemaphoreType.DMA,
        pltpu.SemaphoreType.DMA,
    ])
def _sc_agg0(table, src_hbm, dst_hbm, z_feat, out_hbm,
             src_v, dst_v, rows0, rows1, acc, gsem, ssem):
  c = lax.axis_index("c")
  s = lax.axis_index("s")
  wid = c * NS + s

  pltpu.sync_copy(src_hbm.at[pl.ds(wid * ET, ET)], src_v)
  pltpu.sync_copy(dst_hbm.at[wid], dst_v)

  stripe = pl.ds(s * STRIPE_STEP, STRIPE_LEN)
  pltpu.sync_copy(z_feat.at[stripe], acc.at[stripe])
  plsc.subcore_barrier()

  _pipeline(table, src_v, dst_v, acc, rows0, rows1, gsem, ssem, z_feat, CT)
  plsc.subcore_barrier()

  out_rows = pl.ds(c * N_NODES + s * STRIPE_STEP, STRIPE_LEN)
  pltpu.sync_copy(acc.at[stripe], out_hbm.at[out_rows])


@functools.partial(
    pl.kernel, mesh=_MESH,
    out_type=jax.ShapeDtypeStruct((NW, N_NODES), jnp.float32),
    compiler_params=pltpu.CompilerParams(needs_layout_passes=False),
    scratch_types=[
        pltpu.VMEM((ET,), jnp.int32),
        pltpu.VMEM((N_NODES,), jnp.float32),
    ])
def _sc_counts(dst_hbm, z1d, cnt_out_hbm, dst_v, hist_v):
  c = lax.axis_index("c")
  s = lax.axis_index("s")
  wid = c * NS + s

  pltpu.sync_copy(dst_hbm.at[pl.ds(wid * ET, ET)], dst_v)
  pltpu.sync_copy(z1d, hist_v)

  ones = jnp.ones((16,), jnp.float32)

  def step(i, carry):
    idx = dst_v[pl.ds(i * 16, 16)]
    plsc.addupdate_scatter(hist_v, [idx], ones)
    return carry

  lax.fori_loop(0, ET // 16, step, 0)
  pltpu.sync_copy(hist_v, cnt_out_hbm.at[wid])


@functools.partial(
    pl.kernel, mesh=_MESH,
    out_type=jax.ShapeDtypeStruct((NC * N_NODES, D_FEAT), jnp.float32),
    scratch_types=[
        pltpu.VMEM((ET,), jnp.int32),
        pltpu.VMEM((CT, EC), jnp.int32),
        pltpu.VMEM((EC, D_FEAT), jnp.float32),
        pltpu.VMEM((EC, D_FEAT), jnp.float32),
        pltpu.VMEM_SHARED((N_NODES, D_FEAT), jnp.float32),
        pltpu.SemaphoreType.DMA,
        pltpu.SemaphoreType.DMA,
    ])
def _sc_agg1(table_a, table_b, src_hbm, dst_hbm, z_feat, out_hbm,
             src_v, dst_v, rows0, rows1, acc, gsem, ssem):
  c = lax.axis_index("c")
  s = lax.axis_index("s")

  stripe = pl.ds(s * STRIPE_STEP, STRIPE_LEN)
  pltpu.sync_copy(z_feat.at[stripe], acc.at[stripe])
  plsc.subcore_barrier()

  def run(table):
    for p in range(2):
      pltpu.sync_copy(src_hbm.at[pl.ds((2 * s + p) * ET, ET)], src_v)
      pltpu.sync_copy(dst_hbm.at[2 * s + p], dst_v)
      _pipeline(table, src_v, dst_v, acc, rows0, rows1, gsem, ssem,
                z_feat, CT)

  @pl.when(c == 0)
  def _():
    run(table_a)

  @pl.when(c == 1)
  def _():
    run(table_b)

  plsc.subcore_barrier()

  out_rows = pl.ds(c * N_NODES + s * STRIPE_STEP, STRIPE_LEN)
  pltpu.sync_copy(acc.at[stripe], out_hbm.at[out_rows])


def _tc_cnt_reduce(cntp):
  def body(cntp_r, out_r):
    out_r[:] = jnp.sum(cntp_r[:], axis=0, keepdims=True)
  return pl.pallas_call(
      body,
      out_shape=jax.ShapeDtypeStruct((1, N_NODES), jnp.float32),
  )(cntp)


_BLK = 1000
_GRID = N_NODES // _BLK


def _dense1_body(x, aggp, cnt, wn, ws, b, g, be, h1a, h1b):
  agg = aggp[0] + aggp[1]
  mean = agg / jnp.maximum(cnt, 1.0)
  pre = (jnp.dot(mean, wn[:], preferred_element_type=jnp.float32)
         + jnp.dot(x[:], ws[:], preferred_element_type=jnp.float32) + b[:])
  mu = jnp.mean(pre, axis=-1, keepdims=True)
  var = jnp.mean((pre - mu) ** 2, axis=-1, keepdims=True)
  h = (pre - mu) * lax.rsqrt(var + 1e-5) * g[:] + be[:]
  h = jnp.maximum(h, 0.0)
  h1a[:] = h[:, :D_FEAT]
  h1b[:] = h[:, D_FEAT:]


def _tc_dense1(x, aggp, cnt, wn, ws, b, g, be):
  def body(x_r, aggp_r, cnt_r, wn_r, ws_r, b_r, g_r, be_r, h1a_r, h1b_r):
    _dense1_body(x_r[:], aggp_r[:], cnt_r[:], wn_r, ws_r, b_r, g_r, be_r,
                 h1a_r, h1b_r)
  return pl.pallas_call(
      body,
      grid=(_GRID,),
      in_specs=[
          pl.BlockSpec((_BLK, D_FEAT), lambda i: (i, 0)),
          pl.BlockSpec((NC, _BLK, D_FEAT), lambda i: (0, i, 0)),
          pl.BlockSpec((_BLK, 1), lambda i: (i, 0)),
          pl.BlockSpec((D_FEAT, D_HID), lambda i: (0, 0)),
          pl.BlockSpec((D_FEAT, D_HID), lambda i: (0, 0)),
          pl.BlockSpec((1, D_HID), lambda i: (0, 0)),
          pl.BlockSpec((1, D_HID), lambda i: (0, 0)),
          pl.BlockSpec((1, D_HID), lambda i: (0, 0)),
      ],
      out_specs=[
          pl.BlockSpec((_BLK, D_FEAT), lambda i: (i, 0)),
          pl.BlockSpec((_BLK, D_FEAT), lambda i: (i, 0)),
      ],
      out_shape=[
          jax.ShapeDtypeStruct((N_NODES, D_FEAT), jnp.float32),
          jax.ShapeDtypeStruct((N_NODES, D_FEAT), jnp.float32),
      ],
  )(x, aggp, cnt, wn, ws, b, g, be)


def _tc_dense2(h1a, h1b, agg1, cnt, wn, ws, b, g, be, fcw, fcb):
  def body(h1a_r, h1b_r, agg1_r, cnt_r, wn_r, ws_r, b_r, g_r,
           be_r, fcw_r, fcb_r, out_r):
    h1 = jnp.concatenate([h1a_r[:], h1b_r[:]], axis=1)
    agg = jnp.concatenate([agg1_r[0], agg1_r[1]], axis=1)
    mean = agg / jnp.maximum(cnt_r[:], 1.0)
    pre = (jnp.dot(mean, wn_r[:], preferred_element_type=jnp.float32)
           + jnp.dot(h1, ws_r[:], preferred_element_type=jnp.float32)
           + b_r[:])
    mu = jnp.mean(pre, axis=-1, keepdims=True)
    var = jnp.mean((pre - mu) ** 2, axis=-1, keepdims=True)
    h = (pre - mu) * lax.rsqrt(var + 1e-5) * g_r[:] + be_r[:]
    h = jnp.maximum(h, 0.0)
    res = jnp.dot(h, fcw_r[:], preferred_element_type=jnp.float32) + fcb_r[:]
    out_r[:] = res[:, :N_CLASS]

  return pl.pallas_call(
      body,
      grid=(_GRID,),
      in_specs=[
          pl.BlockSpec((_BLK, D_FEAT), lambda i: (i, 0)),
          pl.BlockSpec((_BLK, D_FEAT), lambda i: (i, 0)),
          pl.BlockSpec((NC, _BLK, D_FEAT), lambda i: (0, i, 0)),
          pl.BlockSpec((_BLK, 1), lambda i: (i, 0)),
          pl.BlockSpec((D_HID, D_HID), lambda i: (0, 0)),
          pl.BlockSpec((D_HID, D_HID), lambda i: (0, 0)),
          pl.BlockSpec((1, D_HID), lambda i: (0, 0)),
          pl.BlockSpec((1, D_HID), lambda i: (0, 0)),
          pl.BlockSpec((1, D_HID), lambda i: (0, 0)),
          pl.BlockSpec((D_HID, 128), lambda i: (0, 0)),
          pl.BlockSpec((1, 128), lambda i: (0, 0)),
      ],
      out_specs=pl.BlockSpec((_BLK, N_CLASS), lambda i: (i, 0)),
      out_shape=jax.ShapeDtypeStruct((N_NODES, N_CLASS), jnp.float32),
  )(h1a, h1b, agg1, cnt, wn, ws, b, g, be, fcw, fcb)


def kernel(x, edge_index, W_nei0, W_self0, b0, g0, be0,
           W_nei1, W_self1, b1, g1, be1, fcW, fcb):
  dst = edge_index[0]
  src = edge_index[1]
  dst2 = dst.reshape(NW, CT, EC)
  z_feat = jnp.zeros((N_NODES, D_FEAT), jnp.float32)
  z1d = jnp.zeros((N_NODES,), jnp.float32)

  agg0p = _sc_agg0(x, src, dst2, z_feat).reshape(NC, N_NODES, D_FEAT)
  cntp = _sc_counts(dst, z1d)
  cnt = _tc_cnt_reduce(cntp).reshape(N_NODES, 1)

  h1a, h1b = _tc_dense1(x, agg0p, cnt, W_nei0, W_self0,
                        b0.reshape(1, D_HID), g0.reshape(1, D_HID),
                        be0.reshape(1, D_HID))

  agg1 = _sc_agg1(h1a, h1b, src, dst2, z_feat).reshape(NC, N_NODES, D_FEAT)

  fcw_pad = jnp.zeros((D_HID, 128), jnp.float32).at[:, :N_CLASS].set(fcW)
  fcb_pad = jnp.zeros((1, 128), jnp.float32).at[:, :N_CLASS].set(fcb)

  return _tc_dense2(h1a, h1b, agg1, cnt, W_nei1, W_self1,
                    b1.reshape(1, D_HID), g1.reshape(1, D_HID),
                    be1.reshape(1, D_HID), fcw_pad, fcb_pad)
